# baseline (speedup 1.0000x reference)
import functools

import jax
import jax.numpy as jnp
from jax import lax
from jax.experimental import pallas as pl
from jax.experimental.pallas import tpu as pltpu
from jax.experimental.pallas import tpu_sc as plsc

L = 16
NC = 2
NS = 16
SC_SPLIT = 0

NEG_BIG = -1e30


def _lane_reduce(vec, op):
    acc = vec[0]
    for i in range(1, L):
        acc = op(acc, vec[i])
    return acc


def _make_sc_pool(B, npg, D, g0):
    NG = B - g0
    assert NG % NC == 0
    NGC = NG // NC
    NT = npg // NS
    DK = D // L
    GRP = NT // L
    PW = D + 2 * L
    mesh = plsc.VectorSubcoreMesh(
        core_axis_name="c", subcore_axis_name="s", num_cores=NC, num_subcores=NS
    )

    @functools.partial(
        pl.kernel,
        out_type=(
            jax.ShapeDtypeStruct((B, D), jnp.float32),
            jax.ShapeDtypeStruct((B, NS, PW), jnp.float32),
        ),
        mesh=mesh,
        scratch_types=[
            pltpu.VMEM((NT, D), jnp.float32),
            pltpu.VMEM((D,), jnp.float32),
            pltpu.VMEM((NT,), jnp.float32),
            pltpu.VMEM((PW,), jnp.float32),
            pltpu.VMEM((NS, PW), jnp.float32),
            pltpu.SemaphoreType.DMA,
        ],
    )
    def sc_pool(x_hbm, wa_hbm, pooled_hbm, part_hbm, xv, wav, sv, pool_acc, pred, sem):
        cid = lax.axis_index("c")
        sid = lax.axis_index("s")
        node0 = sid * NT
        lanes = lax.iota(jnp.int32, L)

        pltpu.async_copy(wa_hbm, wav, sem).wait()

        def graph_body(j, _):
            g = g0 + NC * j + cid
            pltpu.async_copy(x_hbm.at[g, pl.ds(node0, NT)], xv, sem).wait()

            def p1(grp, m):
                base = grp * L
                svec = jnp.zeros((L,), jnp.float32)
                for n in range(L):
                    acc = jnp.zeros((L,), jnp.float32)
                    for k in range(DK):
                        acc = acc + xv[base + n, pl.ds(k * L, L)] * wav[
                            pl.ds(k * L, L)
                        ]
                    s = _lane_reduce(acc, jnp.add)
                    svec = jnp.where(lanes == n, jnp.full((L,), s, jnp.float32), svec)
                sv[pl.ds(base, L)] = svec
                return jnp.maximum(m, svec)

            macc = lax.fori_loop(0, GRP, p1, jnp.full((L,), NEG_BIG, jnp.float32))
            m_t = _lane_reduce(macc, jnp.maximum)
            mv_t = jnp.full((L,), m_t, jnp.float32)

            def p2(grp, carry):
                accs, dacc = carry
                base = grp * L
                wvec = jnp.exp(sv[pl.ds(base, L)] - mv_t)
                accs = list(accs)
                for n in range(L):
                    wb = jnp.full((L,), wvec[n], jnp.float32)
                    for k in range(DK):
                        accs[k] = accs[k] + xv[base + n, pl.ds(k * L, L)] * wb
                return tuple(accs), dacc + wvec

            accs, dacc = lax.fori_loop(
                0,
                GRP,
                p2,
                (tuple(jnp.zeros((L,), jnp.float32) for _ in range(DK)),
                 jnp.zeros((L,), jnp.float32)),
            )
            d_t = _lane_reduce(dacc, jnp.add)
            for k in range(DK):
                pool_acc[pl.ds(k * L, L)] = accs[k]
            pool_acc[pl.ds(D, L)] = mv_t
            pool_acc[pl.ds(D + L, L)] = jnp.full((L,), d_t, jnp.float32)
            pltpu.async_copy(pool_acc, part_hbm.at[g, sid], sem).wait()
            return 0

        lax.fori_loop(0, NGC, graph_body, 0)
        plsc.subcore_barrier()

        @pl.when(sid < NGC)
        def _():
            g = g0 + NC * sid + cid
            pltpu.async_copy(part_hbm.at[g], pred, sem).wait()

            mv = pred[0, pl.ds(D, L)]
            for t in range(1, NS):
                mv = jnp.maximum(mv, pred[t, pl.ds(D, L)])
            dv = jnp.zeros((L,), jnp.float32)
            for t in range(NS):
                dv = dv + jnp.exp(pred[t, pl.ds(D, L)] - mv) * pred[t, pl.ds(D + L, L)]
            rv = jnp.float32(1.0) / dv

            def red(k, _c):
                acc = jnp.zeros((L,), jnp.float32)
                for t in range(NS):
                    acc = acc + pred[t, pl.ds(k * L, L)] * jnp.exp(
                        pred[t, pl.ds(D, L)] - mv
                    )
                pool_acc[pl.ds(k * L, L)] = acc * rv
                return 0

            lax.fori_loop(0, DK, red, 0)
            pltpu.async_copy(pool_acc.at[pl.ds(0, D)], pooled_hbm.at[g], sem).wait()

    return sc_pool


def _tc_pool_body(x_ref, wa_ref, o_ref):
    x = x_ref[0]
    s = jnp.sum(x * wa_ref[...][:, 0][None, :], axis=1)
    m = jnp.max(s)
    e = jnp.exp(s - m)
    w = e / jnp.sum(e)
    o_ref[0] = jnp.sum(x * w[:, None], axis=0)[None, :]


def _proj_body(p_ref, wo_ref, bo_ref, o_ref):
    o_ref[...] = (
        jnp.dot(p_ref[...], wo_ref[...], preferred_element_type=jnp.float32)
        + bo_ref[...][None, :]
    )


def kernel(node_embeddings, batch_num_nodes, Wa, ba, Wo, bo):
    B = batch_num_nodes.shape[0]
    N, D = node_embeddings.shape
    H = Wo.shape[1]
    npg = N // B
    x3 = node_embeddings.reshape(B, npg, D)
    g0 = SC_SPLIT

    pooled_sc, _parts = _make_sc_pool(B, npg, D, g0)(x3, Wa.reshape(D))

    if g0 > 0:
        pooled_tc = pl.pallas_call(
            _tc_pool_body,
            grid=(g0,),
            in_specs=[
                pl.BlockSpec((1, npg, D), lambda i: (i, 0, 0)),
                pl.BlockSpec((D, 1), lambda i: (0, 0)),
            ],
            out_specs=pl.BlockSpec((1, 1, D), lambda i: (i, 0, 0)),
            out_shape=jax.ShapeDtypeStruct((g0, 1, D), jnp.float32),
        )(x3[:g0], Wa).reshape(g0, D)
        pooled = jnp.concatenate([pooled_tc, pooled_sc[g0:]], axis=0)
    else:
        pooled = pooled_sc

    out = pl.pallas_call(
        _proj_body,
        in_specs=[
            pl.BlockSpec((B, D), lambda: (0, 0)),
            pl.BlockSpec((D, H), lambda: (0, 0)),
            pl.BlockSpec((H,), lambda: (0,)),
        ],
        out_specs=pl.BlockSpec((B, H), lambda: (0, 0)),
        out_shape=jax.ShapeDtypeStruct((B, H), jnp.float32),
    )(pooled, Wo, bo)
    return out


# speedup vs baseline: 2.0066x; 2.0066x over previous
import functools

import jax
import jax.numpy as jnp
from jax import lax
from jax.experimental import pallas as pl
from jax.experimental.pallas import tpu as pltpu
from jax.experimental.pallas import tpu_sc as plsc

L = 16
NC = 2
NS = 16
SC_SPLIT = 14

NEG_BIG = -1e30


def _lane_reduce(vec, op):
    acc = vec[0]
    for i in range(1, L):
        acc = op(acc, vec[i])
    return acc


def _make_sc_pool(B, npg, D, g0):
    NG = B - g0
    assert NG % NC == 0
    NGC = NG // NC
    NT = npg // NS
    DK = D // L
    GRP = NT // L
    PW = D + 2 * L
    mesh = plsc.VectorSubcoreMesh(
        core_axis_name="c", subcore_axis_name="s", num_cores=NC, num_subcores=NS
    )

    @functools.partial(
        pl.kernel,
        out_type=(
            jax.ShapeDtypeStruct((B, D), jnp.float32),
            jax.ShapeDtypeStruct((B, NS, PW), jnp.float32),
        ),
        mesh=mesh,
        scratch_types=[
            pltpu.VMEM((NT, D), jnp.float32),
            pltpu.VMEM((D,), jnp.float32),
            pltpu.VMEM((NT,), jnp.float32),
            pltpu.VMEM((PW,), jnp.float32),
            pltpu.VMEM((NS, PW), jnp.float32),
            pltpu.SemaphoreType.DMA,
        ],
    )
    def sc_pool(x_hbm, wa_hbm, pooled_hbm, part_hbm, xv, wav, sv, pool_acc, pred, sem):
        cid = lax.axis_index("c")
        sid = lax.axis_index("s")
        node0 = sid * NT
        lanes = lax.iota(jnp.int32, L)

        pltpu.async_copy(wa_hbm, wav, sem).wait()

        def graph_body(j, _):
            g = g0 + NC * j + cid
            pltpu.async_copy(x_hbm.at[g, pl.ds(node0, NT)], xv, sem).wait()

            def p1(grp, m):
                base = grp * L
                svec = jnp.zeros((L,), jnp.float32)
                for n in range(L):
                    acc = jnp.zeros((L,), jnp.float32)
                    for k in range(DK):
                        acc = acc + xv[base + n, pl.ds(k * L, L)] * wav[
                            pl.ds(k * L, L)
                        ]
                    s = _lane_reduce(acc, jnp.add)
                    svec = jnp.where(lanes == n, jnp.full((L,), s, jnp.float32), svec)
                sv[pl.ds(base, L)] = svec
                return jnp.maximum(m, svec)

            macc = lax.fori_loop(0, GRP, p1, jnp.full((L,), NEG_BIG, jnp.float32))
            m_t = _lane_reduce(macc, jnp.maximum)
            mv_t = jnp.full((L,), m_t, jnp.float32)

            def p2(grp, carry):
                accs, dacc = carry
                base = grp * L
                wvec = jnp.exp(sv[pl.ds(base, L)] - mv_t)
                accs = list(accs)
                for n in range(L):
                    wb = jnp.full((L,), wvec[n], jnp.float32)
                    for k in range(DK):
                        accs[k] = accs[k] + xv[base + n, pl.ds(k * L, L)] * wb
                return tuple(accs), dacc + wvec

            accs, dacc = lax.fori_loop(
                0,
                GRP,
                p2,
                (tuple(jnp.zeros((L,), jnp.float32) for _ in range(DK)),
                 jnp.zeros((L,), jnp.float32)),
            )
            d_t = _lane_reduce(dacc, jnp.add)
            for k in range(DK):
                pool_acc[pl.ds(k * L, L)] = accs[k]
            pool_acc[pl.ds(D, L)] = mv_t
            pool_acc[pl.ds(D + L, L)] = jnp.full((L,), d_t, jnp.float32)
            pltpu.async_copy(pool_acc, part_hbm.at[g, sid], sem).wait()
            return 0

        lax.fori_loop(0, NGC, graph_body, 0)
        plsc.subcore_barrier()

        @pl.when(sid < NGC)
        def _():
            g = g0 + NC * sid + cid
            pltpu.async_copy(part_hbm.at[g], pred, sem).wait()

            mv = pred[0, pl.ds(D, L)]
            for t in range(1, NS):
                mv = jnp.maximum(mv, pred[t, pl.ds(D, L)])
            dv = jnp.zeros((L,), jnp.float32)
            for t in range(NS):
                dv = dv + jnp.exp(pred[t, pl.ds(D, L)] - mv) * pred[t, pl.ds(D + L, L)]
            rv = jnp.float32(1.0) / dv

            def red(k, _c):
                acc = jnp.zeros((L,), jnp.float32)
                for t in range(NS):
                    acc = acc + pred[t, pl.ds(k * L, L)] * jnp.exp(
                        pred[t, pl.ds(D, L)] - mv
                    )
                pool_acc[pl.ds(k * L, L)] = acc * rv
                return 0

            lax.fori_loop(0, DK, red, 0)
            pltpu.async_copy(pool_acc.at[pl.ds(0, D)], pooled_hbm.at[g], sem).wait()

    return sc_pool


def _tc_pool_body(x_ref, wa_ref, o_ref):
    x = x_ref[0]
    s = jnp.sum(x * wa_ref[...][:, 0][None, :], axis=1)
    m = jnp.max(s)
    e = jnp.exp(s - m)
    w = e / jnp.sum(e)
    o_ref[0] = jnp.sum(x * w[:, None], axis=0)[None, :]


def _proj_body(p_ref, wo_ref, bo_ref, o_ref):
    o_ref[...] = (
        jnp.dot(p_ref[...], wo_ref[...], preferred_element_type=jnp.float32)
        + bo_ref[...][None, :]
    )


def kernel(node_embeddings, batch_num_nodes, Wa, ba, Wo, bo):
    B = batch_num_nodes.shape[0]
    N, D = node_embeddings.shape
    H = Wo.shape[1]
    npg = N // B
    x3 = node_embeddings.reshape(B, npg, D)
    g0 = SC_SPLIT

    pooled_sc, _parts = _make_sc_pool(B, npg, D, g0)(x3, Wa.reshape(D))

    if g0 > 0:
        pooled_tc = pl.pallas_call(
            _tc_pool_body,
            grid=(g0,),
            in_specs=[
                pl.BlockSpec((1, npg, D), lambda i: (i, 0, 0)),
                pl.BlockSpec((D, 1), lambda i: (0, 0)),
            ],
            out_specs=pl.BlockSpec((1, 1, D), lambda i: (i, 0, 0)),
            out_shape=jax.ShapeDtypeStruct((g0, 1, D), jnp.float32),
        )(x3[:g0], Wa).reshape(g0, D)
        pooled = jnp.concatenate([pooled_tc, pooled_sc[g0:]], axis=0)
    else:
        pooled = pooled_sc

    out = pl.pallas_call(
        _proj_body,
        in_specs=[
            pl.BlockSpec((B, D), lambda: (0, 0)),
            pl.BlockSpec((D, H), lambda: (0, 0)),
            pl.BlockSpec((H,), lambda: (0,)),
        ],
        out_specs=pl.BlockSpec((B, H), lambda: (0, 0)),
        out_shape=jax.ShapeDtypeStruct((B, H), jnp.float32),
    )(pooled, Wo, bo)
    return out


# speedup vs baseline: 2.2618x; 1.1271x over previous
import functools

import jax
import jax.numpy as jnp
from jax import lax
from jax.experimental import pallas as pl
from jax.experimental.pallas import tpu as pltpu
from jax.experimental.pallas import tpu_sc as plsc

L = 16


def _scores_body(x_ref, wa_ref, ba_ref, o_ref):
    x = x_ref[0]
    s = jnp.sum(x * wa_ref[...][:, 0][None, :], axis=1) + ba_ref[0]
    o_ref[0, 0] = s


def _pool_body(w_ref, x_ref, wo_ref, bo_ref, o_ref, acc_ref):
    g = pl.program_id(0)
    nb = pl.num_programs(0)
    x = x_ref[0]
    w = w_ref[0, 0]
    pooled = jnp.sum(x * w[:, None], axis=0)
    acc_ref[pl.ds(g, 1), :] = pooled[None, :]

    @pl.when(g == nb - 1)
    def _():
        o_ref[...] = (
            jnp.dot(acc_ref[...], wo_ref[...], preferred_element_type=jnp.float32)
            + bo_ref[...][None, :]
        )


def _make_sc_softmax(B, npg):
    nv = npg // L
    mesh = plsc.VectorSubcoreMesh(
        core_axis_name="c", subcore_axis_name="s", num_cores=2, num_subcores=16
    )

    @functools.partial(
        pl.kernel,
        out_type=jax.ShapeDtypeStruct((B, npg), jnp.float32),
        mesh=mesh,
        scratch_types=[
            pltpu.VMEM((npg,), jnp.float32),
            pltpu.VMEM((npg,), jnp.float32),
            pltpu.SemaphoreType.DMA,
        ],
    )
    def sc_softmax(scores_hbm, w_hbm, s_v, e_v, sem):
        wid = lax.axis_index("s") * 2 + lax.axis_index("c")

        def lane_reduce(vec, op):
            acc = vec[0]
            for i in range(1, L):
                acc = op(acc, vec[i])
            return acc

        @pl.when(wid < B)
        def _():
            pltpu.async_copy(scores_hbm.at[wid], s_v, sem).wait()

            def max_body(i, acc):
                return jnp.maximum(acc, s_v[pl.ds(i * L, L)])

            macc = lax.fori_loop(
                0, nv, max_body, jnp.full((L,), -jnp.inf, jnp.float32)
            )
            mv = jnp.full((L,), lane_reduce(macc, jnp.maximum), jnp.float32)

            def exp_body(i, acc):
                e = jnp.exp(s_v[pl.ds(i * L, L)] - mv)
                e_v[pl.ds(i * L, L)] = e
                return acc + e

            dacc = lax.fori_loop(0, nv, exp_body, jnp.zeros((L,), jnp.float32))
            dv = jnp.full((L,), lane_reduce(dacc, jnp.add), jnp.float32)
            rv = jnp.float32(1.0) / dv

            def scale_body(i, _):
                e_v[pl.ds(i * L, L)] = e_v[pl.ds(i * L, L)] * rv
                return 0

            lax.fori_loop(0, nv, scale_body, 0)
            pltpu.async_copy(e_v, w_hbm.at[wid], sem).wait()

    return sc_softmax


def kernel(node_embeddings, batch_num_nodes, Wa, ba, Wo, bo):
    B = batch_num_nodes.shape[0]
    N, D = node_embeddings.shape
    H = Wo.shape[1]
    npg = N // B
    x3 = node_embeddings.reshape(B, npg, D)

    scores = pl.pallas_call(
        _scores_body,
        grid=(B,),
        in_specs=[
            pl.BlockSpec((1, npg, D), lambda i: (i, 0, 0)),
            pl.BlockSpec((D, 1), lambda i: (0, 0)),
            pl.BlockSpec(memory_space=pltpu.SMEM),
        ],
        out_specs=pl.BlockSpec((1, 1, npg), lambda i: (i, 0, 0)),
        out_shape=jax.ShapeDtypeStruct((B, 1, npg), jnp.float32),
    )(x3, Wa, ba).reshape(B, npg)

    w = _make_sc_softmax(B, npg)(scores)

    out = pl.pallas_call(
        _pool_body,
        grid=(B,),
        in_specs=[
            pl.BlockSpec((1, 1, npg), lambda i: (i, 0, 0)),
            pl.BlockSpec((1, npg, D), lambda i: (i, 0, 0)),
            pl.BlockSpec((D, H), lambda i: (0, 0)),
            pl.BlockSpec((H,), lambda i: (0,)),
        ],
        out_specs=pl.BlockSpec((B, H), lambda i: (0, 0)),
        out_shape=jax.ShapeDtypeStruct((B, H), jnp.float32),
        scratch_shapes=[pltpu.VMEM((B, H), jnp.float32)],
    )(w.reshape(B, 1, npg), x3, Wo, bo)
    return out
